# Initial kernel scaffold; baseline (speedup 1.0000x reference)
import jax
import jax.numpy as jnp
from jax.experimental import pallas as pl


def kernel(cls3, reg3, gt3, cls4, reg4, gt4, cls5, reg5, gt5):
    raise NotImplementedError("write your pallas kernel here")


# speedup vs baseline: 7.6765x; 7.6765x over previous
import jax
import jax.numpy as jnp
from jax import lax
from jax.experimental import pallas as pl
from jax.experimental.pallas import tpu as pltpu

_OHEM_RATIO = 3.0


def _f32_from_bits(i):
    return lax.bitcast_convert_type(i, jnp.float32)


def _ce_nll(la, lb, tgt):
    m = jnp.maximum(la, lb)
    lse = m + jnp.log(jnp.exp(la - m) + jnp.exp(lb - m))
    lt = jnp.where(tgt > 0.0, lb, la)
    return lse - lt


def _smooth_l1(d):
    ad = jnp.abs(d)
    return jnp.where(ad < 1.0, 0.5 * d * d, ad - 0.5)


def _dense_level(cls_r, reg_r, gt_r, neg_r):
    rows = pl.ds(0, cls_r.shape[0])
    l0 = cls_r[rows, 0, :]
    l1 = cls_r[rows, 1, :]
    l2 = cls_r[rows, 2, :]
    l3 = cls_r[rows, 3, :]
    tr = gt_r[rows, 0, :]
    tcl = gt_r[rows, 1, :]
    tm = gt_r[rows, 2, :]

    nll_tr = _ce_nll(l0, l1, tr)
    posf = jnp.where(tr * tm > 0.0, 1.0, 0.0).astype(jnp.float32)
    negf = jnp.where((1.0 - tr) * tm > 0.0, 1.0, 0.0).astype(jnp.float32)
    n_pos = jnp.sum(posf)
    n_neg = jnp.sum(negf)
    loss_pos = jnp.sum(posf * nll_tr)
    sum_neg = jnp.sum(negf * nll_tr)
    neg_r[...] = jnp.where(negf > 0.0, nll_tr, -1.0)

    nll_tcl = _ce_nll(l2, l3, tcl)
    s_tcl_pos = jnp.sum(posf * nll_tcl)
    s_tcl_neg = jnp.sum((1.0 - posf) * nll_tcl)

    w = (tr + tcl) * 0.2
    acc = _smooth_l1(reg_r[rows, 0, :] - gt_r[rows, 3, :])
    for c in range(1, 28):
        acc = acc + _smooth_l1(reg_r[rows, c, :] - gt_r[rows, 3 + c, :])
    s_wl = jnp.sum(posf * w * acc)
    return n_pos, n_neg, loss_pos, sum_neg, s_tcl_pos, s_tcl_neg, s_wl


def _topk_sum(neg_r, k, n_neg, sum_neg):
    def _search(_):
        def body(_, lohi):
            lo, hi = lohi
            mid = lo + (hi - lo) // 2
            t = _f32_from_bits(mid)
            cnt = jnp.sum(jnp.where(neg_r[...] >= t, 1.0, 0.0))
            ge = cnt >= k
            return jnp.where(ge, mid, lo), jnp.where(ge, hi, mid)

        lo, _ = lax.fori_loop(
            0, 31, body, (jnp.int32(0), jnp.int32(0x7F800000)))
        v = _f32_from_bits(lo)
        arr = neg_r[...]
        gtm = jnp.where(arr > v, 1.0, 0.0)
        return jnp.sum(arr * gtm) + (k - jnp.sum(gtm)) * v

    return lax.cond(k >= n_neg, lambda _: sum_neg, _search, 0)


def _body(cls3_r, reg3_r, gt3_r, cls4_r, reg4_r, gt4_r, cls5_r, reg5_r, gt5_r,
          out_r, neg3_r, neg4_r, neg5_r):
    ltr = jnp.float32(0.0)
    ltcl = jnp.float32(0.0)
    lwl = jnp.float32(0.0)
    groups = ((cls3_r, reg3_r, gt3_r, neg3_r),
              (cls4_r, reg4_r, gt4_r, neg4_r),
              (cls5_r, reg5_r, gt5_r, neg5_r))
    for cls_r, reg_r, gt_r, neg_r in groups:
        n_pos, n_neg, loss_pos, sum_neg, s_tcl_pos, s_tcl_neg, s_wl = (
            _dense_level(cls_r, reg_r, gt_r, neg_r))
        total = jnp.float32(neg_r.shape[0] * neg_r.shape[1])
        cap = _OHEM_RATIO * n_pos
        nnb = jnp.minimum(n_neg, cap)
        has = n_pos > 0.0
        k = jnp.where(has, nnb, jnp.minimum(n_neg, 100.0))
        denom = jnp.where(has, n_pos + nnb, 100.0)
        s_top = _topk_sum(neg_r, k, n_neg, sum_neg)
        ltr = ltr + (loss_pos + s_top) / denom
        ltcl = ltcl + jnp.where(
            has, s_tcl_pos / n_pos + 0.5 * s_tcl_neg / (total - n_pos), 0.0)
        lwl = lwl + jnp.where(has, s_wl / (n_pos * 28.0), 0.0)

    lane = lax.broadcasted_iota(jnp.int32, (8, 128), 1)
    sub = lax.broadcasted_iota(jnp.int32, (8, 128), 0)
    row0 = sub == 0
    out_r[...] = (jnp.where(row0 & (lane == 0), ltr, 0.0)
                  + jnp.where(row0 & (lane == 1), ltcl, 0.0)
                  + jnp.where(row0 & (lane == 2), lwl, 0.0))


def kernel(cls3, reg3, gt3, cls4, reg4, gt4, cls5, reg5, gt5):
    args = []
    scratch = []
    for c, r, g in ((cls3, reg3, gt3), (cls4, reg4, gt4), (cls5, reg5, gt5)):
        n, _, h, w = c.shape
        hw = h * w
        args += [c.reshape(n, 4, hw), r.reshape(n, 28, hw), g.reshape(n, 31, hw)]
        scratch.append(pltpu.VMEM((n, hw), jnp.float32))
    out = pl.pallas_call(
        _body,
        out_shape=jax.ShapeDtypeStruct((8, 128), jnp.float32),
        scratch_shapes=scratch,
    )(*args)
    return out[0, :3]


# speedup vs baseline: 14.1443x; 1.8425x over previous
import jax
import jax.numpy as jnp
from jax import lax
from jax.experimental import pallas as pl
from jax.experimental.pallas import tpu as pltpu

_OHEM_RATIO = 3.0


def _f32_from_bits(i):
    return lax.bitcast_convert_type(i, jnp.float32)


def _ce_nll(la, lb, tgt):
    m = jnp.maximum(la, lb)
    lse = m + jnp.log(jnp.exp(la - m) + jnp.exp(lb - m))
    lt = jnp.where(tgt > 0.0, lb, la)
    return lse - lt


def _smooth_l1(d):
    ad = jnp.abs(d)
    return jnp.where(ad < 1.0, 0.5 * d * d, ad - 0.5)


def _dense_level(cls_r, reg_r, gt_r, neg_r):
    l0 = cls_r[:, 0]
    l1 = cls_r[:, 1]
    l2 = cls_r[:, 2]
    l3 = cls_r[:, 3]
    tr = gt_r[:, 0]
    tcl = gt_r[:, 1]
    tm = gt_r[:, 2]

    nll_tr = _ce_nll(l0, l1, tr)
    posf = jnp.where(tr * tm > 0.0, 1.0, 0.0).astype(jnp.float32)
    negf = jnp.where((1.0 - tr) * tm > 0.0, 1.0, 0.0).astype(jnp.float32)
    n_pos = jnp.sum(posf)
    n_neg = jnp.sum(negf)
    loss_pos = jnp.sum(posf * nll_tr)
    sum_neg = jnp.sum(negf * nll_tr)
    neg_r[...] = jnp.where(negf > 0.0, nll_tr, -1.0)

    nll_tcl = _ce_nll(l2, l3, tcl)
    s_tcl_pos = jnp.sum(posf * nll_tcl)
    s_tcl_neg = jnp.sum((1.0 - posf) * nll_tcl)

    w = (tr + tcl) * 0.2
    acc = _smooth_l1(reg_r[:, 0] - gt_r[:, 3])
    for c in range(1, 28):
        acc = acc + _smooth_l1(reg_r[:, c] - gt_r[:, 3 + c])
    s_wl = jnp.sum(posf * w * acc)
    return n_pos, n_neg, loss_pos, sum_neg, s_tcl_pos, s_tcl_neg, s_wl


def _topk_sum(neg_r, k, n_neg, sum_neg):
    def _search(_):
        def body(_, lohi):
            lo, hi = lohi
            mid = lo + (hi - lo) // 2
            t = _f32_from_bits(mid)
            cnt = jnp.sum(jnp.where(neg_r[...] >= t, 1.0, 0.0))
            ge = cnt >= k
            return jnp.where(ge, mid, lo), jnp.where(ge, hi, mid)

        lo, _ = lax.fori_loop(
            0, 31, body, (jnp.int32(0), jnp.int32(0x7F800000)))
        v = _f32_from_bits(lo)
        arr = neg_r[...]
        gtm = jnp.where(arr > v, 1.0, 0.0)
        return jnp.sum(arr * gtm) + (k - jnp.sum(gtm)) * v

    return lax.cond(k >= n_neg, lambda _: sum_neg, _search, 0)


def _body(cls3_r, reg3_r, gt3_r, cls4_r, reg4_r, gt4_r, cls5_r, reg5_r, gt5_r,
          out_r, neg3_r, neg4_r, neg5_r):
    ltr = jnp.float32(0.0)
    ltcl = jnp.float32(0.0)
    lwl = jnp.float32(0.0)
    groups = ((cls3_r, reg3_r, gt3_r, neg3_r),
              (cls4_r, reg4_r, gt4_r, neg4_r),
              (cls5_r, reg5_r, gt5_r, neg5_r))
    for cls_r, reg_r, gt_r, neg_r in groups:
        n_pos, n_neg, loss_pos, sum_neg, s_tcl_pos, s_tcl_neg, s_wl = (
            _dense_level(cls_r, reg_r, gt_r, neg_r))
        total = jnp.float32(neg_r.shape[0] * neg_r.shape[1] * neg_r.shape[2])
        cap = _OHEM_RATIO * n_pos
        nnb = jnp.minimum(n_neg, cap)
        has = n_pos > 0.0
        k = jnp.where(has, nnb, jnp.minimum(n_neg, 100.0))
        denom = jnp.where(has, n_pos + nnb, 100.0)
        s_top = _topk_sum(neg_r, k, n_neg, sum_neg)
        ltr = ltr + (loss_pos + s_top) / denom
        ltcl = ltcl + jnp.where(
            has, s_tcl_pos / n_pos + 0.5 * s_tcl_neg / (total - n_pos), 0.0)
        lwl = lwl + jnp.where(has, s_wl / (n_pos * 28.0), 0.0)

    lane = lax.broadcasted_iota(jnp.int32, (8, 128), 1)
    sub = lax.broadcasted_iota(jnp.int32, (8, 128), 0)
    row0 = sub == 0
    out_r[...] = (jnp.where(row0 & (lane == 0), ltr, 0.0)
                  + jnp.where(row0 & (lane == 1), ltcl, 0.0)
                  + jnp.where(row0 & (lane == 2), lwl, 0.0))


def kernel(cls3, reg3, gt3, cls4, reg4, gt4, cls5, reg5, gt5):
    args = []
    scratch = []
    for c, r, g in ((cls3, reg3, gt3), (cls4, reg4, gt4), (cls5, reg5, gt5)):
        n, _, h, w = c.shape
        s = (h * w) // 128
        args += [c.reshape(n, 4, s, 128), r.reshape(n, 28, s, 128),
                 g.reshape(n, 31, s, 128)]
        scratch.append(pltpu.VMEM((n, s, 128), jnp.float32))
    out = pl.pallas_call(
        _body,
        out_shape=jax.ShapeDtypeStruct((8, 128), jnp.float32),
        scratch_shapes=scratch,
    )(*args)
    return out[0, :3]


# speedup vs baseline: 17.5176x; 1.2385x over previous
import jax
import jax.numpy as jnp
from jax import lax
from jax.experimental import pallas as pl
from jax.experimental.pallas import tpu as pltpu

_OHEM_RATIO = 3.0
_NSTAT = 7


def _f32_from_bits(i):
    return lax.bitcast_convert_type(i, jnp.float32)


def _ce_nll(la, lb, tgt):
    m = jnp.maximum(la, lb)
    lse = m + jnp.log(jnp.exp(la - m) + jnp.exp(lb - m))
    lt = jnp.where(tgt > 0.0, lb, la)
    return lse - lt


def _smooth_l1(d):
    ad = jnp.abs(d)
    return jnp.where(ad < 1.0, 0.5 * d * d, ad - 0.5)


def _dense_step(cls_r, reg_r, gt_r, neg_r, i):
    l0 = cls_r[0, 0]
    l1 = cls_r[0, 1]
    l2 = cls_r[0, 2]
    l3 = cls_r[0, 3]
    tr = gt_r[0, 0]
    tcl = gt_r[0, 1]
    tm = gt_r[0, 2]

    nll_tr = _ce_nll(l0, l1, tr)
    posf = jnp.where(tr * tm > 0.0, 1.0, 0.0).astype(jnp.float32)
    negf = jnp.where((1.0 - tr) * tm > 0.0, 1.0, 0.0).astype(jnp.float32)
    n_pos = jnp.sum(posf)
    n_neg = jnp.sum(negf)
    loss_pos = jnp.sum(posf * nll_tr)
    sum_neg = jnp.sum(negf * nll_tr)
    neg_r[i] = jnp.where(negf > 0.0, nll_tr, -1.0)

    nll_tcl = _ce_nll(l2, l3, tcl)
    s_tcl_pos = jnp.sum(posf * nll_tcl)
    s_tcl_neg = jnp.sum((1.0 - posf) * nll_tcl)

    w = (tr + tcl) * 0.2
    acc = _smooth_l1(reg_r[0, 0] - gt_r[0, 3])
    for c in range(1, 28):
        acc = acc + _smooth_l1(reg_r[0, c] - gt_r[0, 3 + c])
    s_wl = jnp.sum(posf * w * acc)
    return n_pos, n_neg, loss_pos, sum_neg, s_tcl_pos, s_tcl_neg, s_wl


def _topk_sum(neg_r, k, n_neg, sum_neg):
    def _search(_):
        def body(_, lohi):
            lo, hi = lohi
            mid = lo + (hi - lo) // 2
            t = _f32_from_bits(mid)
            cnt = jnp.sum(jnp.where(neg_r[...] >= t, 1.0, 0.0))
            ge = cnt >= k
            return jnp.where(ge, mid, lo), jnp.where(ge, hi, mid)

        lo, _ = lax.fori_loop(
            0, 31, body, (jnp.int32(0), jnp.int32(0x7F800000)))
        v = _f32_from_bits(lo)
        arr = neg_r[...]
        gtm = jnp.where(arr > v, 1.0, 0.0)
        return jnp.sum(arr * gtm) + (k - jnp.sum(gtm)) * v

    return lax.cond(k >= n_neg, lambda _: sum_neg, _search, 0)


def _body(cls3_r, reg3_r, gt3_r, cls4_r, reg4_r, gt4_r, cls5_r, reg5_r, gt5_r,
          out_r, neg3_r, neg4_r, neg5_r, acc_r):
    i = pl.program_id(0)
    groups = ((cls3_r, reg3_r, gt3_r, neg3_r),
              (cls4_r, reg4_r, gt4_r, neg4_r),
              (cls5_r, reg5_r, gt5_r, neg5_r))
    for lvl, (cls_r, reg_r, gt_r, neg_r) in enumerate(groups):
        part = _dense_step(cls_r, reg_r, gt_r, neg_r, i)
        for j, p in enumerate(part):
            prev = jnp.where(i > 0, acc_r[lvl, j], 0.0)
            acc_r[lvl, j] = prev + p

    @pl.when(i == pl.num_programs(0) - 1)
    def _finalize():
        ltr = jnp.float32(0.0)
        ltcl = jnp.float32(0.0)
        lwl = jnp.float32(0.0)
        for lvl, (_, _, _, neg_r) in enumerate(groups):
            n_pos = acc_r[lvl, 0]
            n_neg = acc_r[lvl, 1]
            loss_pos = acc_r[lvl, 2]
            sum_neg = acc_r[lvl, 3]
            s_tcl_pos = acc_r[lvl, 4]
            s_tcl_neg = acc_r[lvl, 5]
            s_wl = acc_r[lvl, 6]
            total = jnp.float32(
                neg_r.shape[0] * neg_r.shape[1] * neg_r.shape[2])
            cap = _OHEM_RATIO * n_pos
            nnb = jnp.minimum(n_neg, cap)
            has = n_pos > 0.0
            k = jnp.where(has, nnb, jnp.minimum(n_neg, 100.0))
            denom = jnp.where(has, n_pos + nnb, 100.0)
            s_top = _topk_sum(neg_r, k, n_neg, sum_neg)
            ltr = ltr + (loss_pos + s_top) / denom
            ltcl = ltcl + jnp.where(
                has, s_tcl_pos / n_pos + 0.5 * s_tcl_neg / (total - n_pos),
                0.0)
            lwl = lwl + jnp.where(has, s_wl / (n_pos * 28.0), 0.0)

        lane = lax.broadcasted_iota(jnp.int32, (8, 128), 1)
        sub = lax.broadcasted_iota(jnp.int32, (8, 128), 0)
        row0 = sub == 0
        out_r[...] = (jnp.where(row0 & (lane == 0), ltr, 0.0)
                      + jnp.where(row0 & (lane == 1), ltcl, 0.0)
                      + jnp.where(row0 & (lane == 2), lwl, 0.0))


def kernel(cls3, reg3, gt3, cls4, reg4, gt4, cls5, reg5, gt5):
    args = []
    in_specs = []
    scratch = []
    n = cls3.shape[0]
    for c, r, g in ((cls3, reg3, gt3), (cls4, reg4, gt4), (cls5, reg5, gt5)):
        _, _, h, w = c.shape
        s = (h * w) // 128
        args += [c.reshape(n, 4, s, 128), r.reshape(n, 28, s, 128),
                 g.reshape(n, 31, s, 128)]
        for ch in (4, 28, 31):
            in_specs.append(
                pl.BlockSpec((1, ch, s, 128), lambda i: (i, 0, 0, 0)))
        scratch.append(pltpu.VMEM((n, s, 128), jnp.float32))
    scratch.append(pltpu.SMEM((3, _NSTAT), jnp.float32))
    out = pl.pallas_call(
        _body,
        grid=(n,),
        in_specs=in_specs,
        out_specs=pl.BlockSpec((8, 128), lambda i: (0, 0)),
        out_shape=jax.ShapeDtypeStruct((8, 128), jnp.float32),
        scratch_shapes=scratch,
        compiler_params=pltpu.CompilerParams(
            dimension_semantics=("arbitrary",)),
    )(*args)
    return out[0, :3]
